# baseline (speedup 1.0000x reference)
import functools

import jax
import jax.numpy as jnp
from jax import lax
from jax.experimental import pallas as pl
from jax.experimental.pallas import tpu as pltpu
from jax.experimental.pallas import tpu_sc as plsc

NC = 2
NS = 16
NW = NC * NS
L = 16

B, H, V, D = 16384, 200, 25, 32
DT = D // 8
BB = B // 128
SLABS = H * DT
SLAB_ELEMS = BB * 8 * 128
SLABS_PER_W = SLABS // NW
WT_PER_CHUNK = 16
CHUNK_ELEMS = WT_PER_CHUNK * 1024
CHUNKS = BB // WT_PER_CHUNK
B_PER_CHUNK = WT_PER_CHUNK * 128
TSTRIDE = V * D + 1


def kernel(positions, table):
    pos_t = jnp.maximum(positions, jnp.int32(0)).T
    table_flat = table.reshape(V * D)

    mesh = plsc.VectorSubcoreMesh(
        core_axis_name="c", subcore_axis_name="s", num_cores=NC, num_subcores=NS
    )

    @functools.partial(
        pl.kernel,
        out_type=jax.ShapeDtypeStruct((H, DT, SLAB_ELEMS), jnp.float32),
        mesh=mesh,
        scratch_types=[
            pltpu.VMEM((V * D,), jnp.float32),
            pltpu.VMEM((L * TSTRIDE,), jnp.float32),
            pltpu.VMEM((B,), jnp.int32),
            pltpu.VMEM((2, CHUNK_ELEMS), jnp.float32),
            pltpu.SemaphoreType.DMA((2,)),
        ],
        compiler_params=pltpu.CompilerParams(use_tc_tiling_on_sc=False,
                                             needs_layout_passes=False),
    )
    def gather_kernel(pos_hbm, table_hbm, out_hbm, tstage_v, table_v,
                      idxrow_v, stage_v, wsem):
        cid = lax.axis_index("c")
        sid = lax.axis_index("s")
        wid = sid * NC + cid

        pltpu.sync_copy(table_hbm, tstage_v)
        iota16 = lax.iota(jnp.int32, L)
        for g in range(V * D // L):
            v = tstage_v[pl.ds(g * L, L)]
            for i in range(L):
                plsc.store_scatter(
                    table_v, [iota16 + (i * TSTRIDE + g * L)], v)

        lanevec = iota16 * TSTRIDE

        def wb_descriptor(h, dt, c, buf):
            return pltpu.make_async_copy(
                stage_v.at[buf],
                out_hbm.at[h, dt, pl.ds(c * CHUNK_ELEMS, CHUNK_ELEMS)],
                wsem.at[buf])

        def do_slab(j, carry):
            s = wid * SLABS_PER_W + j
            h = s // DT
            dt = s % DT

            @pl.when(jnp.logical_or(j == 0, dt == 0))
            def _():
                pltpu.sync_copy(pos_hbm.at[h], idxrow_v)

            dbase = dt * 8

            def do_chunk(c, carry2):
                buf = c % 2

                @pl.when(j * CHUNKS + c >= 2)
                def _():
                    wb_descriptor(h, dt, c, buf).wait()

                cbase = c * B_PER_CHUNK

                def do_wt(w, carry3):
                    wbase = w * 1024
                    ibase = cbase + w * 128
                    vals_a = []
                    for lg in range(4):
                        idxv = idxrow_v[pl.ds(ibase + lg * L, L)]
                        base = idxv * D + (lanevec + dbase)
                        for d in range(8):
                            vals_a.append(
                                (wbase + d * 128 + lg * L,
                                 plsc.load_gather(table_v, [base + d])))
                    vals_b = []
                    k = 0
                    for lg in range(4, 8):
                        idxv = idxrow_v[pl.ds(ibase + lg * L, L)]
                        base = idxv * D + (lanevec + dbase)
                        for d in range(8):
                            vals_b.append(
                                (wbase + d * 128 + lg * L,
                                 plsc.load_gather(table_v, [base + d])))
                            off_a, v_a = vals_a[k]
                            stage_v[buf, pl.ds(off_a, L)] = v_a
                            k += 1
                    for off, v in vals_b:
                        stage_v[buf, pl.ds(off, L)] = v
                    return carry3

                lax.fori_loop(0, WT_PER_CHUNK, do_wt, 0, unroll=False)

                pltpu.async_copy(
                    stage_v.at[buf],
                    out_hbm.at[h, dt, pl.ds(c * CHUNK_ELEMS, CHUNK_ELEMS)],
                    wsem.at[buf])
                return carry2

            lax.fori_loop(0, CHUNKS, do_chunk, carry, unroll=False)
            return carry

        lax.fori_loop(0, SLABS_PER_W, do_slab, jnp.int32(0), unroll=False)

        last = wid * SLABS_PER_W + SLABS_PER_W - 1
        for c in (CHUNKS - 2, CHUNKS - 1):
            wb_descriptor(last // DT, last % DT, c, c % 2).wait()

    p = gather_kernel(pos_t, table_flat)
    return (p.reshape(H, DT, BB, 8, 128)
            .transpose(2, 4, 0, 1, 3).reshape(B, H, D))


# speedup vs baseline: 1.0524x; 1.0524x over previous
import functools

import jax
import jax.numpy as jnp
from jax import lax
from jax.experimental import pallas as pl
from jax.experimental.pallas import tpu as pltpu
from jax.experimental.pallas import tpu_sc as plsc

NC = 2
NS = 16
NW = NC * NS
L = 16

B, H, V, D = 16384, 200, 25, 32
DT = D // 8
BB = B // 128
SLABS = H * DT
SLAB_ELEMS = BB * 8 * 128
SLABS_PER_W = SLABS // NW
WT_PER_CHUNK = 32
CHUNK_ELEMS = WT_PER_CHUNK * 1024
CHUNKS = BB // WT_PER_CHUNK
B_PER_CHUNK = WT_PER_CHUNK * 128
TSTRIDE = V * D + 1


def kernel(positions, table):
    pos_t = jnp.maximum(positions, jnp.int32(0)).T
    table_flat = table.reshape(V * D)

    mesh = plsc.VectorSubcoreMesh(
        core_axis_name="c", subcore_axis_name="s", num_cores=NC, num_subcores=NS
    )

    @functools.partial(
        pl.kernel,
        out_type=jax.ShapeDtypeStruct((H, DT, SLAB_ELEMS), jnp.float32),
        mesh=mesh,
        scratch_types=[
            pltpu.VMEM((V * D,), jnp.float32),
            pltpu.VMEM((L * TSTRIDE,), jnp.float32),
            pltpu.VMEM((B,), jnp.int32),
            pltpu.VMEM((2, CHUNK_ELEMS), jnp.float32),
            pltpu.SemaphoreType.DMA((2,)),
        ],
        compiler_params=pltpu.CompilerParams(use_tc_tiling_on_sc=False,
                                             needs_layout_passes=False),
    )
    def gather_kernel(pos_hbm, table_hbm, out_hbm, tstage_v, table_v,
                      idxrow_v, stage_v, wsem):
        cid = lax.axis_index("c")
        sid = lax.axis_index("s")
        wid = sid * NC + cid

        pltpu.sync_copy(table_hbm, tstage_v)
        iota16 = lax.iota(jnp.int32, L)
        for g in range(V * D // L):
            v = tstage_v[pl.ds(g * L, L)]
            for i in range(L):
                plsc.store_scatter(
                    table_v, [iota16 + (i * TSTRIDE + g * L)], v)

        lanevec = iota16 * TSTRIDE

        def wb_descriptor(h, dt, c, buf):
            return pltpu.make_async_copy(
                stage_v.at[buf],
                out_hbm.at[h, dt, pl.ds(c * CHUNK_ELEMS, CHUNK_ELEMS)],
                wsem.at[buf])

        def do_slab(j, carry):
            s = wid * SLABS_PER_W + j
            h = s // DT
            dt = s % DT

            @pl.when(jnp.logical_or(j == 0, dt == 0))
            def _():
                pltpu.sync_copy(pos_hbm.at[h], idxrow_v)

            dbase = dt * 8

            def do_chunk(c, carry2):
                buf = c % 2

                @pl.when(j * CHUNKS + c >= 2)
                def _():
                    wb_descriptor(h, dt, c, buf).wait()

                cbase = c * B_PER_CHUNK

                def do_wt(w, carry3):
                    wbase = w * 1024
                    ibase = cbase + w * 128
                    vals = []
                    for lgrp in range(8):
                        idxv = idxrow_v[pl.ds(ibase + lgrp * L, L)]
                        base = idxv * D + (lanevec + dbase)
                        for d in range(8):
                            vals.append(
                                (wbase + d * 128 + lgrp * L,
                                 plsc.load_gather(table_v, [base + d])))
                    for off, v in vals:
                        stage_v[buf, pl.ds(off, L)] = v
                    return carry3

                lax.fori_loop(0, WT_PER_CHUNK, do_wt, 0, unroll=False)

                pltpu.async_copy(
                    stage_v.at[buf],
                    out_hbm.at[h, dt, pl.ds(c * CHUNK_ELEMS, CHUNK_ELEMS)],
                    wsem.at[buf])
                return carry2

            lax.fori_loop(0, CHUNKS, do_chunk, carry, unroll=False)
            return carry

        lax.fori_loop(0, SLABS_PER_W, do_slab, jnp.int32(0), unroll=False)

        last = wid * SLABS_PER_W + SLABS_PER_W - 1
        for c in (CHUNKS - 2, CHUNKS - 1):
            wb_descriptor(last // DT, last % DT, c, c % 2).wait()

    p = gather_kernel(pos_t, table_flat)
    return (p.reshape(H, DT, BB, 8, 128)
            .transpose(2, 4, 0, 1, 3).reshape(B, H, D))


# speedup vs baseline: 1.0921x; 1.0378x over previous
import functools

import jax
import jax.numpy as jnp
from jax import lax
from jax.experimental import pallas as pl
from jax.experimental.pallas import tpu as pltpu
from jax.experimental.pallas import tpu_sc as plsc

NC = 2
NS = 16
NW = NC * NS
L = 16

B, H, V, D = 16384, 200, 25, 32
DT = D // 8
BB = B // 128
SLABS = H * DT
SLAB_ELEMS = BB * 8 * 128
SLABS_PER_W = SLABS // NW
WT_PER_CHUNK = 32
CHUNK_ELEMS = WT_PER_CHUNK * 1024
CHUNKS = BB // WT_PER_CHUNK
B_PER_CHUNK = WT_PER_CHUNK * 128
TSTRIDE = V * D + 1


def kernel(positions, table):
    pos_t = jnp.maximum(positions, jnp.int32(0)).T
    table_flat = table.reshape(V * D)

    mesh = plsc.VectorSubcoreMesh(
        core_axis_name="c", subcore_axis_name="s", num_cores=NC, num_subcores=NS
    )

    @functools.partial(
        pl.kernel,
        out_type=jax.ShapeDtypeStruct((H, DT, SLAB_ELEMS), jnp.float32),
        mesh=mesh,
        scratch_types=[
            pltpu.VMEM((V * D,), jnp.float32),
            pltpu.VMEM((L * TSTRIDE,), jnp.float32),
            pltpu.VMEM((B,), jnp.int32),
            pltpu.VMEM((2, CHUNK_ELEMS), jnp.float32),
            pltpu.SemaphoreType.DMA((2,)),
        ],
        compiler_params=pltpu.CompilerParams(use_tc_tiling_on_sc=False,
                                             needs_layout_passes=False),
    )
    def gather_kernel(pos_hbm, table_hbm, out_hbm, tstage_v, table_v,
                      idxrow_v, stage_v, wsem):
        cid = lax.axis_index("c")
        sid = lax.axis_index("s")
        wid = sid * NC + cid

        pltpu.sync_copy(table_hbm, tstage_v)
        iota16 = lax.iota(jnp.int32, L)
        for g in range(V * D // L):
            v = tstage_v[pl.ds(g * L, L)]
            for i in range(L):
                plsc.store_scatter(
                    table_v, [iota16 + (i * TSTRIDE + g * L)], v)

        lanevec = iota16 * TSTRIDE

        def wb_descriptor(h, dt, c, buf):
            return pltpu.make_async_copy(
                stage_v.at[buf],
                out_hbm.at[h, dt, pl.ds(c * CHUNK_ELEMS, CHUNK_ELEMS)],
                wsem.at[buf])

        def do_slab(j, carry):
            s = wid * SLABS_PER_W + j
            h = s // DT
            dt = s % DT

            @pl.when(jnp.logical_or(j == 0, dt == 0))
            def _():
                pltpu.sync_copy(pos_hbm.at[h], idxrow_v)

            dbase = dt * 8

            def do_chunk(c, carry2):
                buf = c % 2

                @pl.when(j * CHUNKS + c >= 2)
                def _():
                    wb_descriptor(h, dt, c, buf).wait()

                cbase = c * B_PER_CHUNK

                def do_wt(w, carry3):
                    wbase = w * 1024
                    ibase = cbase + w * 128
                    vals = []
                    for lgrp in range(8):
                        idxv = idxrow_v[pl.ds(ibase + lgrp * L, L)]
                        base = idxv * D + (lanevec + dbase)
                        for d in range(8):
                            vals.append(
                                (wbase + d * 128 + lgrp * L,
                                 plsc.load_gather(table_v, [base + d])))
                    for off, v in vals:
                        stage_v[buf, pl.ds(off, L)] = v
                    return carry3

                lax.fori_loop(0, WT_PER_CHUNK, do_wt, 0, unroll=2)

                pltpu.async_copy(
                    stage_v.at[buf],
                    out_hbm.at[h, dt, pl.ds(c * CHUNK_ELEMS, CHUNK_ELEMS)],
                    wsem.at[buf])
                return carry2

            lax.fori_loop(0, CHUNKS, do_chunk, carry, unroll=False)
            return carry

        lax.fori_loop(0, SLABS_PER_W, do_slab, jnp.int32(0), unroll=False)

        last = wid * SLABS_PER_W + SLABS_PER_W - 1
        for c in (CHUNKS - 2, CHUNKS - 1):
            wb_descriptor(last // DT, last % DT, c, c % 2).wait()

    p = gather_kernel(pos_t, table_flat)
    return (p.reshape(H, DT, BB, 8, 128)
            .transpose(2, 4, 0, 1, 3).reshape(B, H, D))
